# Initial kernel scaffold; baseline (speedup 1.0000x reference)
import jax
import jax.numpy as jnp
from jax.experimental import pallas as pl


def kernel(atom_type, formal_charge, num_H, aromaticity, hybridization, chiral, bond_type, stereo, conjugated, in_ring, graph_distance, W_atom_type, W_formal_charge, W_num_H, W_aromaticity, W_hybridization, W_chiral, W_bond_type, W_stereo, W_conjugated, W_in_ring, W_graph_distance):
    raise NotImplementedError("write your pallas kernel here")


# speedup vs baseline: 2.5305x; 2.5305x over previous
import functools

import jax
import jax.numpy as jnp
from jax import lax
from jax.experimental import pallas as pl
from jax.experimental.pallas import tpu as pltpu
from jax.experimental.pallas import tpu_sc as plsc

N_ATOMS = 50000
N_BONDS = 800000
D_ATOM = 128
D_BOND = 64

NC = 2
NS = 16
NW = NC * NS
L = 16

A_CHUNK = 80
B_CHUNK = 128
A_NCHUNK = N_ATOMS // A_CHUNK
B_NCHUNK = N_BONDS // B_CHUNK

_f32 = jnp.float32
_i32 = jnp.int32


def _build_fused_tables(Wat, War, Wfc, WnH, Why, Wch, Wbt, Wst, Wcj, Wir,
                        Wgd):
    def body(a, r, fc, nh, hy, ch, b, s, c, i, fa1, fa2, fa3, fbc):
        fa1[...] = a[...] + r[...]
        fa2[...] = fc[...] + nh[...]
        fa3[...] = hy[...] + ch[...]
        fbc[...] = b[...] + s[...] + c[...] + i[...]

    a_e = jnp.repeat(Wat, 2, axis=0)
    r_e = jnp.tile(War, (100, 1))
    fc_e = jnp.repeat(Wfc, 9, axis=0)
    nh_e = jnp.tile(WnH, (8, 1))
    hy_e = jnp.repeat(Why, 4, axis=0)
    ch_e = jnp.tile(Wch, (8, 1))
    zpad = jnp.zeros((256, D_BOND), _f32)
    gd_e = jnp.pad(Wgd, ((0, 256 - 32), (0, 0)))
    b_e = jnp.concatenate([jnp.repeat(Wbt, 32, axis=0), gd_e], axis=1)
    s_e = jnp.concatenate(
        [jnp.tile(jnp.repeat(Wst, 4, axis=0), (8, 1)), zpad], axis=1)
    c_e = jnp.concatenate(
        [jnp.tile(jnp.repeat(Wcj, 2, axis=0), (64, 1)), zpad], axis=1)
    i_e = jnp.concatenate([jnp.tile(Wir, (128, 1)), zpad], axis=1)

    return pl.pallas_call(
        body,
        out_shape=[
            jax.ShapeDtypeStruct((200, D_ATOM), _f32),
            jax.ShapeDtypeStruct((72, D_ATOM), _f32),
            jax.ShapeDtypeStruct((32, D_ATOM), _f32),
            jax.ShapeDtypeStruct((256, 128), _f32),
        ],
    )(a_e, r_e, fc_e, nh_e, hy_e, ch_e, b_e, s_e, c_e, i_e)


def _sc_lookup(at, fc, nh, ar, hy, ch, bt, st, cj, ir, gd, fa1, fa2, fa3,
               fbc):
    mesh = plsc.VectorSubcoreMesh(core_axis_name="c", subcore_axis_name="s")

    @functools.partial(
        pl.kernel,
        out_type=[
            jax.ShapeDtypeStruct((N_ATOMS, D_ATOM), _f32),
            jax.ShapeDtypeStruct((N_BONDS, D_BOND), _f32),
        ],
        mesh=mesh,
        compiler_params=pltpu.CompilerParams(needs_layout_passes=False),
        scratch_types=[
            pltpu.VMEM((200, D_ATOM), _f32),
            pltpu.VMEM((72, D_ATOM), _f32),
            pltpu.VMEM((32, D_ATOM), _f32),
            pltpu.VMEM((256, 128), _f32),
            [pltpu.VMEM((A_CHUNK,), _i32) for _ in range(6)],
            pltpu.VMEM((A_CHUNK, D_ATOM), _f32),
            [[pltpu.VMEM((B_CHUNK,), _i32) for _ in range(5)]
             for _ in range(2)],
            pltpu.VMEM((2, B_CHUNK, D_BOND), _f32),
            pltpu.SemaphoreType.DMA,
            pltpu.SemaphoreType.DMA,
            [pltpu.SemaphoreType.DMA, pltpu.SemaphoreType.DMA],
            [pltpu.SemaphoreType.DMA, pltpu.SemaphoreType.DMA],
        ],
    )
    def k(at_h, fc_h, nh_h, ar_h, hy_h, ch_h,
          bt_h, st_h, cj_h, ir_h, gd_h,
          fa1_h, fa2_h, fa3_h, fbc_h,
          atom_out, bond_out,
          fa1_v, fa2_v, fa3_v, fbc_v,
          ai_v, ao_v, bi_v, bo_v,
          s_ai, s_ao, s_bi, s_bo):
        wid = lax.axis_index("s") * NC + lax.axis_index("c")

        pltpu.sync_copy(fa1_h, fa1_v)
        pltpu.sync_copy(fa2_h, fa2_v)
        pltpu.sync_copy(fa3_h, fa3_v)
        pltpu.sync_copy(fbc_h, fbc_v)

        a_idx_h = (at_h, fc_h, nh_h, ar_h, hy_h, ch_h)

        def a_start_in(kk):
            off = (wid + NW * kk) * A_CHUNK
            for j, h in enumerate(a_idx_h):
                pltpu.async_copy(h.at[pl.ds(off, A_CHUNK)], ai_v[j], s_ai)

        def a_wait_in():
            for j, h in enumerate(a_idx_h):
                pltpu.make_async_copy(h.at[pl.ds(0, A_CHUNK)], ai_v[j],
                                      s_ai).wait()

        def a_compute():
            @pl.loop(0, A_CHUNK // L)
            def _(g):
                s = g * L
                v_at = ai_v[0][pl.ds(s, L)]
                v_fc = ai_v[1][pl.ds(s, L)]
                v_nh = ai_v[2][pl.ds(s, L)]
                v_ar = ai_v[3][pl.ds(s, L)]
                v_hy = ai_v[4][pl.ds(s, L)]
                v_ch = ai_v[5][pl.ds(s, L)]
                r1 = v_at * 2 + v_ar
                r2 = (v_fc + 1) * 9 + v_nh
                r3 = v_hy * 4 + v_ch
                rows = s + lax.iota(_i32, L)
                for dd in range(D_ATOM):
                    cd = jnp.full((L,), dd, _i32)
                    v = (plsc.load_gather(fa1_v, [r1, cd])
                         + plsc.load_gather(fa2_v, [r2, cd])
                         + plsc.load_gather(fa3_v, [r3, cd]))
                    plsc.store_scatter(ao_v, [rows, cd], v)

        def a_start_out(kk):
            off = (wid + NW * kk) * A_CHUNK
            pltpu.async_copy(ao_v, atom_out.at[pl.ds(off, A_CHUNK)], s_ao)

        def a_wait_out():
            pltpu.make_async_copy(ao_v, atom_out.at[pl.ds(0, A_CHUNK)],
                                  s_ao).wait()

        n_a = (A_NCHUNK - wid + NW - 1) // NW

        @pl.loop(0, n_a)
        def _(kk):
            a_start_in(kk)
            a_wait_in()

            @pl.when(kk >= 1)
            def _():
                a_wait_out()

            a_compute()
            a_start_out(kk)

        @pl.when(n_a >= 1)
        def _():
            a_wait_out()

        b_idx_h = (bt_h, st_h, cj_h, ir_h, gd_h)

        def b_start_in(b, kk):
            off = (wid + NW * kk) * B_CHUNK
            for j, h in enumerate(b_idx_h):
                pltpu.async_copy(h.at[pl.ds(off, B_CHUNK)], bi_v[b][j],
                                 s_bi[b])

        def b_wait_in(b):
            for j, h in enumerate(b_idx_h):
                pltpu.make_async_copy(h.at[pl.ds(0, B_CHUNK)], bi_v[b][j],
                                      s_bi[b]).wait()

        def b_compute(b, kk):
            @pl.loop(0, B_CHUNK // L)
            def _(g):
                s = g * L
                v_bt = bi_v[b][0][pl.ds(s, L)]
                v_st = bi_v[b][1][pl.ds(s, L)]
                v_cj = bi_v[b][2][pl.ds(s, L)]
                v_ir = bi_v[b][3][pl.ds(s, L)]
                v_gd = bi_v[b][4][pl.ds(s, L)]
                r1 = ((v_bt * 8 + v_st) * 2 + v_cj) * 2 + v_ir
                rows = s + lax.iota(_i32, L)
                for dd in range(D_BOND):
                    cd = jnp.full((L,), dd, _i32)
                    cdg = jnp.full((L,), D_BOND + dd, _i32)
                    v = (plsc.load_gather(fbc_v, [r1, cd])
                         + plsc.load_gather(fbc_v, [v_gd, cdg]))
                    plsc.store_scatter(bo_v.at[b], [rows, cd], v)

        def b_start_out(b, kk):
            off = (wid + NW * kk) * B_CHUNK
            pltpu.async_copy(bo_v.at[b], bond_out.at[pl.ds(off, B_CHUNK)],
                             s_bo[b])

        def b_wait_out(b):
            pltpu.make_async_copy(bo_v.at[b], bond_out.at[pl.ds(0, B_CHUNK)],
                                  s_bo[b]).wait()

        n_b = (B_NCHUNK - wid + NW - 1) // NW

        @pl.when(n_b >= 1)
        def _():
            b_start_in(0, 0)

        @pl.when(n_b >= 2)
        def _():
            b_start_in(1, 1)

        @pl.loop(0, n_b, step=2)
        def _(k0):
            for b in range(2):
                kk = k0 + b

                @pl.when(kk < n_b)
                def _():
                    b_wait_in(b)

                    @pl.when(kk >= 2)
                    def _():
                        b_wait_out(b)

                    b_compute(b, kk)
                    b_start_out(b, kk)

                    @pl.when(kk + 2 < n_b)
                    def _():
                        b_start_in(b, kk + 2)

        @pl.when(n_b >= 1)
        def _():
            b_wait_out(0)

        @pl.when(n_b >= 2)
        def _():
            b_wait_out(1)

    return k(at, fc, nh, ar, hy, ch, bt, st, cj, ir, gd, fa1, fa2, fa3, fbc)


def kernel(atom_type, formal_charge, num_H, aromaticity, hybridization,
           chiral, bond_type, stereo, conjugated, in_ring, graph_distance,
           W_atom_type, W_formal_charge, W_num_H, W_aromaticity,
           W_hybridization, W_chiral, W_bond_type, W_stereo, W_conjugated,
           W_in_ring, W_graph_distance):
    fa1, fa2, fa3, fbc = _build_fused_tables(
        W_atom_type, W_aromaticity, W_formal_charge, W_num_H,
        W_hybridization, W_chiral, W_bond_type, W_stereo, W_conjugated,
        W_in_ring, W_graph_distance)
    atom_emb, bond_emb = _sc_lookup(
        atom_type, formal_charge, num_H, aromaticity, hybridization, chiral,
        bond_type, stereo, conjugated, in_ring, graph_distance,
        fa1, fa2, fa3, fbc)
    return (atom_emb, bond_emb)


# speedup vs baseline: 9.7264x; 3.8437x over previous
import functools

import jax
import jax.numpy as jnp
from jax import lax
from jax.experimental import pallas as pl
from jax.experimental.pallas import tpu as pltpu
from jax.experimental.pallas import tpu_sc as plsc

N_ATOMS = 50000
N_BONDS = 800000
D_ATOM = 128
D_BOND = 64

NC = 2
NS = 16
NW = NC * NS
L = 16

A_CHUNK = 80
B_CHUNK = 128
A_NCHUNK = N_ATOMS // A_CHUNK
B_NCHUNK = N_BONDS // B_CHUNK

_f32 = jnp.float32
_i32 = jnp.int32


def _build_fused_tables(Wat, War, Wfc, WnH, Why, Wch, Wbt, Wst, Wcj, Wir,
                        Wgd):
    def body(a, r, fc, nh, hy, ch, b, s, c, i, fa1, fa2, fa3, fbc):
        fa1[...] = a[...] + r[...]
        fa2[...] = fc[...] + nh[...]
        fa3[...] = hy[...] + ch[...]
        fbc[...] = b[...] + s[...] + c[...] + i[...]

    a_e = jnp.repeat(Wat, 2, axis=0)
    r_e = jnp.tile(War, (100, 1))
    fc_e = jnp.repeat(Wfc, 9, axis=0)
    nh_e = jnp.tile(WnH, (8, 1))
    hy_e = jnp.repeat(Why, 4, axis=0)
    ch_e = jnp.tile(Wch, (8, 1))
    zpad = jnp.zeros((256, D_BOND), _f32)
    gd_e = jnp.pad(Wgd, ((0, 256 - 32), (0, 0)))
    b_e = jnp.concatenate([jnp.repeat(Wbt, 32, axis=0), gd_e], axis=1)
    s_e = jnp.concatenate(
        [jnp.tile(jnp.repeat(Wst, 4, axis=0), (8, 1)), zpad], axis=1)
    c_e = jnp.concatenate(
        [jnp.tile(jnp.repeat(Wcj, 2, axis=0), (64, 1)), zpad], axis=1)
    i_e = jnp.concatenate([jnp.tile(Wir, (128, 1)), zpad], axis=1)

    return pl.pallas_call(
        body,
        out_shape=[
            jax.ShapeDtypeStruct((200, D_ATOM), _f32),
            jax.ShapeDtypeStruct((72, D_ATOM), _f32),
            jax.ShapeDtypeStruct((32, D_ATOM), _f32),
            jax.ShapeDtypeStruct((256, 128), _f32),
        ],
    )(a_e, r_e, fc_e, nh_e, hy_e, ch_e, b_e, s_e, c_e, i_e)


def _sc_lookup(at, fc, nh, ar, hy, ch, bt, st, cj, ir, gd, fa1, fa2, fa3,
               fbc):
    mesh = plsc.VectorSubcoreMesh(core_axis_name="c", subcore_axis_name="s")

    @functools.partial(
        pl.kernel,
        out_type=[
            jax.ShapeDtypeStruct((N_ATOMS, D_ATOM), _f32),
            jax.ShapeDtypeStruct((N_BONDS, D_BOND), _f32),
        ],
        mesh=mesh,
        compiler_params=pltpu.CompilerParams(needs_layout_passes=False),
        scratch_types=[
            pltpu.VMEM((200, D_ATOM), _f32),
            pltpu.VMEM((72, D_ATOM), _f32),
            pltpu.VMEM((32, D_ATOM), _f32),
            pltpu.VMEM((256, 128), _f32),
            [pltpu.VMEM((A_CHUNK,), _i32) for _ in range(6)],
            pltpu.VMEM((A_CHUNK, D_ATOM), _f32),
            [[pltpu.VMEM((B_CHUNK,), _i32) for _ in range(5)]
             for _ in range(2)],
            pltpu.VMEM((2, B_CHUNK, D_BOND), _f32),
            pltpu.SemaphoreType.DMA,
            pltpu.SemaphoreType.DMA,
            [pltpu.SemaphoreType.DMA, pltpu.SemaphoreType.DMA],
            [pltpu.SemaphoreType.DMA, pltpu.SemaphoreType.DMA],
        ],
    )
    def k(at_h, fc_h, nh_h, ar_h, hy_h, ch_h,
          bt_h, st_h, cj_h, ir_h, gd_h,
          fa1_h, fa2_h, fa3_h, fbc_h,
          atom_out, bond_out,
          fa1_v, fa2_v, fa3_v, fbc_v,
          ai_v, ao_v, bi_v, bo_v,
          s_ai, s_ao, s_bi, s_bo):
        wid = lax.axis_index("s") * NC + lax.axis_index("c")

        pltpu.sync_copy(fa1_h, fa1_v)
        pltpu.sync_copy(fa2_h, fa2_v)
        pltpu.sync_copy(fa3_h, fa3_v)
        pltpu.sync_copy(fbc_h, fbc_v)

        a_idx_h = (at_h, fc_h, nh_h, ar_h, hy_h, ch_h)

        def a_start_in(kk):
            off = (wid + NW * kk) * A_CHUNK
            for j, h in enumerate(a_idx_h):
                pltpu.async_copy(h.at[pl.ds(off, A_CHUNK)], ai_v[j], s_ai)

        def a_wait_in():
            for j, h in enumerate(a_idx_h):
                pltpu.make_async_copy(h.at[pl.ds(0, A_CHUNK)], ai_v[j],
                                      s_ai).wait()

        def a_compute():
            @pl.loop(0, A_CHUNK // L)
            def _(g):
                s = g * L
                v_at = ai_v[0][pl.ds(s, L)]
                v_fc = ai_v[1][pl.ds(s, L)]
                v_nh = ai_v[2][pl.ds(s, L)]
                v_ar = ai_v[3][pl.ds(s, L)]
                v_hy = ai_v[4][pl.ds(s, L)]
                v_ch = ai_v[5][pl.ds(s, L)]
                r1 = v_at * 2 + v_ar
                r2 = (v_fc + 1) * 9 + v_nh
                r3 = v_hy * 4 + v_ch
                for j in range(L):
                    r1j, r2j, r3j = r1[j], r2[j], r3[j]
                    for c in range(D_ATOM // L):
                        cs = pl.ds(c * L, L)
                        ao_v[s + j, cs] = (fa1_v[r1j, cs] + fa2_v[r2j, cs]
                                           + fa3_v[r3j, cs])

        def a_start_out(kk):
            off = (wid + NW * kk) * A_CHUNK
            pltpu.async_copy(ao_v, atom_out.at[pl.ds(off, A_CHUNK)], s_ao)

        def a_wait_out():
            pltpu.make_async_copy(ao_v, atom_out.at[pl.ds(0, A_CHUNK)],
                                  s_ao).wait()

        n_a = (A_NCHUNK - wid + NW - 1) // NW

        @pl.loop(0, n_a)
        def _(kk):
            a_start_in(kk)
            a_wait_in()

            @pl.when(kk >= 1)
            def _():
                a_wait_out()

            a_compute()
            a_start_out(kk)

        @pl.when(n_a >= 1)
        def _():
            a_wait_out()

        b_idx_h = (bt_h, st_h, cj_h, ir_h, gd_h)

        def b_start_in(b, kk):
            off = (wid + NW * kk) * B_CHUNK
            for j, h in enumerate(b_idx_h):
                pltpu.async_copy(h.at[pl.ds(off, B_CHUNK)], bi_v[b][j],
                                 s_bi[b])

        def b_wait_in(b):
            for j, h in enumerate(b_idx_h):
                pltpu.make_async_copy(h.at[pl.ds(0, B_CHUNK)], bi_v[b][j],
                                      s_bi[b]).wait()

        def b_compute(b, kk):
            @pl.loop(0, B_CHUNK // L)
            def _(g):
                s = g * L
                v_bt = bi_v[b][0][pl.ds(s, L)]
                v_st = bi_v[b][1][pl.ds(s, L)]
                v_cj = bi_v[b][2][pl.ds(s, L)]
                v_ir = bi_v[b][3][pl.ds(s, L)]
                v_gd = bi_v[b][4][pl.ds(s, L)]
                r1 = ((v_bt * 8 + v_st) * 2 + v_cj) * 2 + v_ir
                for j in range(L):
                    r1j, gdj = r1[j], v_gd[j]
                    for c in range(D_BOND // L):
                        cs = pl.ds(c * L, L)
                        bo_v[b, s + j, cs] = (
                            fbc_v[r1j, cs]
                            + fbc_v[gdj, pl.ds(D_BOND + c * L, L)])

        def b_start_out(b, kk):
            off = (wid + NW * kk) * B_CHUNK
            pltpu.async_copy(bo_v.at[b], bond_out.at[pl.ds(off, B_CHUNK)],
                             s_bo[b])

        def b_wait_out(b):
            pltpu.make_async_copy(bo_v.at[b], bond_out.at[pl.ds(0, B_CHUNK)],
                                  s_bo[b]).wait()

        n_b = (B_NCHUNK - wid + NW - 1) // NW

        @pl.when(n_b >= 1)
        def _():
            b_start_in(0, 0)

        @pl.when(n_b >= 2)
        def _():
            b_start_in(1, 1)

        @pl.loop(0, n_b, step=2)
        def _(k0):
            for b in range(2):
                kk = k0 + b

                @pl.when(kk < n_b)
                def _():
                    b_wait_in(b)

                    @pl.when(kk >= 2)
                    def _():
                        b_wait_out(b)

                    b_compute(b, kk)
                    b_start_out(b, kk)

                    @pl.when(kk + 2 < n_b)
                    def _():
                        b_start_in(b, kk + 2)

        @pl.when(n_b >= 1)
        def _():
            b_wait_out(0)

        @pl.when(n_b >= 2)
        def _():
            b_wait_out(1)

    return k(at, fc, nh, ar, hy, ch, bt, st, cj, ir, gd, fa1, fa2, fa3, fbc)


def kernel(atom_type, formal_charge, num_H, aromaticity, hybridization,
           chiral, bond_type, stereo, conjugated, in_ring, graph_distance,
           W_atom_type, W_formal_charge, W_num_H, W_aromaticity,
           W_hybridization, W_chiral, W_bond_type, W_stereo, W_conjugated,
           W_in_ring, W_graph_distance):
    fa1, fa2, fa3, fbc = _build_fused_tables(
        W_atom_type, W_aromaticity, W_formal_charge, W_num_H,
        W_hybridization, W_chiral, W_bond_type, W_stereo, W_conjugated,
        W_in_ring, W_graph_distance)
    atom_emb, bond_emb = _sc_lookup(
        atom_type, formal_charge, num_H, aromaticity, hybridization, chiral,
        bond_type, stereo, conjugated, in_ring, graph_distance,
        fa1, fa2, fa3, fbc)
    return (atom_emb, bond_emb)


# speedup vs baseline: 10.3568x; 1.0648x over previous
import functools

import jax
import jax.numpy as jnp
from jax import lax
from jax.experimental import pallas as pl
from jax.experimental.pallas import tpu as pltpu
from jax.experimental.pallas import tpu_sc as plsc

N_ATOMS = 50000
N_BONDS = 800000
D_ATOM = 128
D_BOND = 64

NC = 2
NS = 16
NW = NC * NS
L = 16

A_CHUNK = 80
B_CHUNK = 128
A_NCHUNK = N_ATOMS // A_CHUNK
B_NCHUNK = N_BONDS // B_CHUNK

_f32 = jnp.float32
_i32 = jnp.int32


def _build_fused_tables(Wat, War, Wfc, WnH, Why, Wch, Wbt, Wst, Wcj, Wir,
                        Wgd):
    def body(a, r, fc, nh, hy, ch, b, s, c, i, fa1, fa2, fa3, fbc):
        fa1[...] = a[...] + r[...]
        fa2[...] = fc[...] + nh[...]
        fa3[...] = hy[...] + ch[...]
        fbc[...] = b[...] + s[...] + c[...] + i[...]

    a_e = jnp.repeat(Wat, 2, axis=0)
    r_e = jnp.tile(War, (100, 1))
    fc_e = jnp.repeat(Wfc, 9, axis=0)
    nh_e = jnp.tile(WnH, (8, 1))
    hy_e = jnp.repeat(Why, 4, axis=0)
    ch_e = jnp.tile(Wch, (8, 1))
    zpad = jnp.zeros((256, D_BOND), _f32)
    gd_e = jnp.pad(Wgd, ((0, 256 - 32), (0, 0)))
    b_e = jnp.concatenate([jnp.repeat(Wbt, 32, axis=0), gd_e], axis=1)
    s_e = jnp.concatenate(
        [jnp.tile(jnp.repeat(Wst, 4, axis=0), (8, 1)), zpad], axis=1)
    c_e = jnp.concatenate(
        [jnp.tile(jnp.repeat(Wcj, 2, axis=0), (64, 1)), zpad], axis=1)
    i_e = jnp.concatenate([jnp.tile(Wir, (128, 1)), zpad], axis=1)

    return pl.pallas_call(
        body,
        out_shape=[
            jax.ShapeDtypeStruct((200, D_ATOM), _f32),
            jax.ShapeDtypeStruct((72, D_ATOM), _f32),
            jax.ShapeDtypeStruct((32, D_ATOM), _f32),
            jax.ShapeDtypeStruct((256, 128), _f32),
        ],
    )(a_e, r_e, fc_e, nh_e, hy_e, ch_e, b_e, s_e, c_e, i_e)


def _sc_lookup(at, fc, nh, ar, hy, ch, bt, st, cj, ir, gd, fa1, fa2, fa3,
               fbc):
    mesh = plsc.VectorSubcoreMesh(core_axis_name="c", subcore_axis_name="s")

    @functools.partial(
        pl.kernel,
        out_type=[
            jax.ShapeDtypeStruct((N_ATOMS, D_ATOM), _f32),
            jax.ShapeDtypeStruct((N_BONDS, D_BOND), _f32),
        ],
        mesh=mesh,
        compiler_params=pltpu.CompilerParams(needs_layout_passes=False),
        scratch_types=[
            pltpu.VMEM((200, D_ATOM), _f32),
            pltpu.VMEM((72, D_ATOM), _f32),
            pltpu.VMEM((32, D_ATOM), _f32),
            pltpu.VMEM((256, 128), _f32),
            [pltpu.VMEM((A_CHUNK,), _i32) for _ in range(6)],
            pltpu.VMEM((A_CHUNK, D_ATOM), _f32),
            [[pltpu.VMEM((B_CHUNK,), _i32) for _ in range(5)]
             for _ in range(2)],
            pltpu.VMEM((2, B_CHUNK, D_BOND), _f32),
            pltpu.SemaphoreType.DMA,
            pltpu.SemaphoreType.DMA,
            [pltpu.SemaphoreType.DMA, pltpu.SemaphoreType.DMA],
            [pltpu.SemaphoreType.DMA, pltpu.SemaphoreType.DMA],
        ],
    )
    def k(at_h, fc_h, nh_h, ar_h, hy_h, ch_h,
          bt_h, st_h, cj_h, ir_h, gd_h,
          fa1_h, fa2_h, fa3_h, fbc_h,
          atom_out, bond_out,
          fa1_v, fa2_v, fa3_v, fbc_v,
          ai_v, ao_v, bi_v, bo_v,
          s_ai, s_ao, s_bi, s_bo):
        wid = lax.axis_index("s") * NC + lax.axis_index("c")

        pltpu.sync_copy(fa1_h, fa1_v)
        pltpu.sync_copy(fa2_h, fa2_v)
        pltpu.sync_copy(fa3_h, fa3_v)
        pltpu.sync_copy(fbc_h, fbc_v)

        a_idx_h = (at_h, fc_h, nh_h, ar_h, hy_h, ch_h)

        def a_start_in(kk):
            off = (wid + NW * kk) * A_CHUNK
            for j, h in enumerate(a_idx_h):
                pltpu.async_copy(h.at[pl.ds(off, A_CHUNK)], ai_v[j], s_ai)

        def a_wait_in():
            for j, h in enumerate(a_idx_h):
                pltpu.make_async_copy(h.at[pl.ds(0, A_CHUNK)], ai_v[j],
                                      s_ai).wait()

        def a_compute():
            @plsc.parallel_loop(0, A_CHUNK // L, unroll=2)
            def _(g):
                s = g * L
                v_at = ai_v[0][pl.ds(s, L)]
                v_fc = ai_v[1][pl.ds(s, L)]
                v_nh = ai_v[2][pl.ds(s, L)]
                v_ar = ai_v[3][pl.ds(s, L)]
                v_hy = ai_v[4][pl.ds(s, L)]
                v_ch = ai_v[5][pl.ds(s, L)]
                r1 = v_at * 2 + v_ar
                r2 = (v_fc + 1) * 9 + v_nh
                r3 = v_hy * 4 + v_ch
                for j in range(L):
                    r1j, r2j, r3j = r1[j], r2[j], r3[j]
                    for c in range(D_ATOM // L):
                        cs = pl.ds(c * L, L)
                        ao_v[s + j, cs] = (fa1_v[r1j, cs] + fa2_v[r2j, cs]
                                           + fa3_v[r3j, cs])

        def a_start_out(kk):
            off = (wid + NW * kk) * A_CHUNK
            pltpu.async_copy(ao_v, atom_out.at[pl.ds(off, A_CHUNK)], s_ao)

        def a_wait_out():
            pltpu.make_async_copy(ao_v, atom_out.at[pl.ds(0, A_CHUNK)],
                                  s_ao).wait()

        n_a = (A_NCHUNK - wid + NW - 1) // NW

        @pl.loop(0, n_a)
        def _(kk):
            a_start_in(kk)
            a_wait_in()

            @pl.when(kk >= 1)
            def _():
                a_wait_out()

            a_compute()
            a_start_out(kk)

        @pl.when(n_a >= 1)
        def _():
            a_wait_out()

        b_idx_h = (bt_h, st_h, cj_h, ir_h, gd_h)

        def b_start_in(b, kk):
            off = (wid + NW * kk) * B_CHUNK
            for j, h in enumerate(b_idx_h):
                pltpu.async_copy(h.at[pl.ds(off, B_CHUNK)], bi_v[b][j],
                                 s_bi[b])

        def b_wait_in(b):
            for j, h in enumerate(b_idx_h):
                pltpu.make_async_copy(h.at[pl.ds(0, B_CHUNK)], bi_v[b][j],
                                      s_bi[b]).wait()

        def b_compute(b, kk):
            @plsc.parallel_loop(0, B_CHUNK // L, unroll=2)
            def _(g):
                s = g * L
                v_bt = bi_v[b][0][pl.ds(s, L)]
                v_st = bi_v[b][1][pl.ds(s, L)]
                v_cj = bi_v[b][2][pl.ds(s, L)]
                v_ir = bi_v[b][3][pl.ds(s, L)]
                v_gd = bi_v[b][4][pl.ds(s, L)]
                r1 = ((v_bt * 8 + v_st) * 2 + v_cj) * 2 + v_ir
                for j in range(L):
                    r1j, gdj = r1[j], v_gd[j]
                    for c in range(D_BOND // L):
                        cs = pl.ds(c * L, L)
                        bo_v[b, s + j, cs] = (
                            fbc_v[r1j, cs]
                            + fbc_v[gdj, pl.ds(D_BOND + c * L, L)])

        def b_start_out(b, kk):
            off = (wid + NW * kk) * B_CHUNK
            pltpu.async_copy(bo_v.at[b], bond_out.at[pl.ds(off, B_CHUNK)],
                             s_bo[b])

        def b_wait_out(b):
            pltpu.make_async_copy(bo_v.at[b], bond_out.at[pl.ds(0, B_CHUNK)],
                                  s_bo[b]).wait()

        n_b = (B_NCHUNK - wid + NW - 1) // NW

        @pl.when(n_b >= 1)
        def _():
            b_start_in(0, 0)

        @pl.when(n_b >= 2)
        def _():
            b_start_in(1, 1)

        @pl.loop(0, n_b, step=2)
        def _(k0):
            for b in range(2):
                kk = k0 + b

                @pl.when(kk < n_b)
                def _():
                    b_wait_in(b)

                    @pl.when(kk >= 2)
                    def _():
                        b_wait_out(b)

                    b_compute(b, kk)
                    b_start_out(b, kk)

                    @pl.when(kk + 2 < n_b)
                    def _():
                        b_start_in(b, kk + 2)

        @pl.when(n_b >= 1)
        def _():
            b_wait_out(0)

        @pl.when(n_b >= 2)
        def _():
            b_wait_out(1)

    return k(at, fc, nh, ar, hy, ch, bt, st, cj, ir, gd, fa1, fa2, fa3, fbc)


def kernel(atom_type, formal_charge, num_H, aromaticity, hybridization,
           chiral, bond_type, stereo, conjugated, in_ring, graph_distance,
           W_atom_type, W_formal_charge, W_num_H, W_aromaticity,
           W_hybridization, W_chiral, W_bond_type, W_stereo, W_conjugated,
           W_in_ring, W_graph_distance):
    fa1, fa2, fa3, fbc = _build_fused_tables(
        W_atom_type, W_aromaticity, W_formal_charge, W_num_H,
        W_hybridization, W_chiral, W_bond_type, W_stereo, W_conjugated,
        W_in_ring, W_graph_distance)
    atom_emb, bond_emb = _sc_lookup(
        atom_type, formal_charge, num_H, aromaticity, hybridization, chiral,
        bond_type, stereo, conjugated, in_ring, graph_distance,
        fa1, fa2, fa3, fbc)
    return (atom_emb, bond_emb)


# speedup vs baseline: 10.6827x; 1.0315x over previous
import functools

import jax
import jax.numpy as jnp
from jax import lax
from jax.experimental import pallas as pl
from jax.experimental.pallas import tpu as pltpu
from jax.experimental.pallas import tpu_sc as plsc

N_ATOMS = 50000
N_BONDS = 800000
D_ATOM = 128
D_BOND = 64

NC = 2
NS = 16
NW = NC * NS
L = 16

A_CHUNK = 80
B_CHUNK = 128
A_NCHUNK = N_ATOMS // A_CHUNK
B_NCHUNK = N_BONDS // B_CHUNK

_f32 = jnp.float32
_i32 = jnp.int32


def _expand(w, inner, outer):
    return jnp.tile(jnp.repeat(w, inner, axis=0), (outer, 1))


def _build_fused_tables(Wat, Wfc, WnH, War, Why, Wch, Wbt, Wst, Wcj, Wir,
                        Wgd):
    def body(b, s, c, i, g, fc, nh, ar, hy, ch, fb, fa5):
        left = b[...] + s[...] + c[...] + i[...] + g[...]
        fb[...] = jnp.concatenate(
            [left, jnp.zeros((8192, D_BOND), _f32)], axis=1)
        fa5[...] = fc[...] + nh[...] + ar[...] + hy[...] + ch[...]

    b_e = _expand(Wbt, 1024, 1)
    s_e = _expand(Wst, 128, 8)
    c_e = _expand(Wcj, 64, 64)
    i_e = _expand(Wir, 32, 128)
    g_e = _expand(Wgd, 1, 256)
    fc_e = _expand(Wfc, 576, 1)
    nh_e = _expand(WnH, 64, 8)
    ar_e = _expand(War, 32, 72)
    hy_e = _expand(Why, 4, 144)
    ch_e = _expand(Wch, 1, 1152)

    return pl.pallas_call(
        body,
        out_shape=[
            jax.ShapeDtypeStruct((8192, 2 * D_BOND), _f32),
            jax.ShapeDtypeStruct((4608, D_ATOM), _f32),
        ],
    )(b_e, s_e, c_e, i_e, g_e, fc_e, nh_e, ar_e, hy_e, ch_e)


def _sc_lookup(at, fc, nh, ar, hy, ch, bt, st, cj, ir, gd, FB, FA5, Wat):
    mesh = plsc.VectorSubcoreMesh(core_axis_name="c", subcore_axis_name="s")

    @functools.partial(
        pl.kernel,
        out_type=[
            jax.ShapeDtypeStruct((N_ATOMS, D_ATOM), _f32),
            jax.ShapeDtypeStruct((N_BONDS, D_BOND), _f32),
        ],
        mesh=mesh,
        compiler_params=pltpu.CompilerParams(needs_layout_passes=False),
        scratch_types=[
            [[pltpu.VMEM((A_CHUNK,), _i32) for _ in range(6)]
             for _ in range(2)],
            [pltpu.VMEM((A_CHUNK,), _i32) for _ in range(2)],
            pltpu.VMEM((2, A_CHUNK, D_ATOM), _f32),
            pltpu.VMEM((2, A_CHUNK, D_ATOM), _f32),
            [[pltpu.VMEM((B_CHUNK,), _i32) for _ in range(5)]
             for _ in range(2)],
            [pltpu.VMEM((B_CHUNK,), _i32) for _ in range(2)],
            pltpu.VMEM((2, B_CHUNK, 2 * D_BOND), _f32),
            pltpu.VMEM((2, B_CHUNK, D_BOND), _f32),
            [pltpu.SemaphoreType.DMA, pltpu.SemaphoreType.DMA],
            [pltpu.SemaphoreType.DMA, pltpu.SemaphoreType.DMA],
            [pltpu.SemaphoreType.DMA, pltpu.SemaphoreType.DMA],
            [pltpu.SemaphoreType.DMA, pltpu.SemaphoreType.DMA],
            [pltpu.SemaphoreType.DMA, pltpu.SemaphoreType.DMA],
            [pltpu.SemaphoreType.DMA, pltpu.SemaphoreType.DMA],
        ],
    )
    def k(at_h, fc_h, nh_h, ar_h, hy_h, ch_h,
          bt_h, st_h, cj_h, ir_h, gd_h,
          FB_h, FA5_h, Wat_h,
          atom_out, bond_out,
          ai_v, ra_v, ao_v, ab_v, bi_v, rb_v, bo_v, bc_v,
          s_ai, s_ag, s_ao, s_bi, s_bg, s_bo):
        wid = lax.axis_index("s") * NC + lax.axis_index("c")

        def run_pipeline(n, start_in, wait_in, compute, start_gather,
                         wait_gather, post, start_out, wait_out):
            @pl.when(n >= 1)
            def _():
                start_in(0, 0)

            @pl.loop(0, n, step=2)
            def _(k0):
                for b in range(2):
                    kk = k0 + b
                    o = 1 - b

                    @pl.when(kk < n)
                    def _():
                        wait_in(b)

                        @pl.when(kk + 1 < n)
                        def _():
                            start_in(o, kk + 1)

                        compute(b)

                        @pl.when(kk >= 1)
                        def _():
                            wait_gather(o)
                            post(o)
                            start_out(o, kk - 1)

                        @pl.when(kk >= 2)
                        def _():
                            wait_out(b)

                        start_gather(b)

            last = (n - 1) % 2

            @pl.when(n >= 1)
            def _():
                for b in range(2):
                    @pl.when(last == b)
                    def _():
                        wait_gather(b)
                        post(b)
                        start_out(b, n - 1)

                wait_out(0)

            @pl.when(n >= 2)
            def _():
                wait_out(1)

        a_idx_h = (at_h, fc_h, nh_h, ar_h, hy_h, ch_h)

        def a_start_in(b, kk):
            off = (wid + NW * kk) * A_CHUNK
            for j, h in enumerate(a_idx_h):
                pltpu.async_copy(h.at[pl.ds(off, A_CHUNK)], ai_v[b][j],
                                 s_ai[b])

        def a_wait_in(b):
            for j, h in enumerate(a_idx_h):
                pltpu.make_async_copy(h.at[pl.ds(0, A_CHUNK)], ai_v[b][j],
                                      s_ai[b]).wait()

        def a_compute(b):
            @pl.loop(0, A_CHUNK // L)
            def _(g):
                s = pl.ds(g * L, L)
                v_fc = ai_v[b][1][s]
                v_nh = ai_v[b][2][s]
                v_ar = ai_v[b][3][s]
                v_hy = ai_v[b][4][s]
                v_ch = ai_v[b][5][s]
                ra = ((((v_fc + 1) * 9 + v_nh) * 2 + v_ar) * 8
                      + v_hy) * 4 + v_ch
                ra_v[b][s] = ra

        def a_start_gather(b):
            pltpu.async_copy(FA5_h.at[ra_v[b]], ao_v.at[b], s_ag[b])
            pltpu.async_copy(Wat_h.at[ai_v[b][0]], ab_v.at[b], s_ag[b])

        def a_wait_gather(b):
            pltpu.make_async_copy(FA5_h.at[ra_v[b]], ao_v.at[b],
                                  s_ag[b]).wait()
            pltpu.make_async_copy(Wat_h.at[ai_v[b][0]], ab_v.at[b],
                                  s_ag[b]).wait()

        def a_post(b):
            @pl.loop(0, A_CHUNK)
            def _(r):
                for c in range(D_ATOM // L):
                    cs = pl.ds(c * L, L)
                    plsc.addupdate(ao_v.at[b, r, cs], ab_v[b, r, cs])

        def a_start_out(b, kk):
            off = (wid + NW * kk) * A_CHUNK
            pltpu.async_copy(ao_v.at[b], atom_out.at[pl.ds(off, A_CHUNK)],
                             s_ao[b])

        def a_wait_out(b):
            pltpu.make_async_copy(ao_v.at[b], atom_out.at[pl.ds(0, A_CHUNK)],
                                  s_ao[b]).wait()

        n_a = (A_NCHUNK - wid + NW - 1) // NW
        run_pipeline(n_a, a_start_in, a_wait_in, a_compute, a_start_gather,
                     a_wait_gather, a_post, a_start_out, a_wait_out)

        b_idx_h = (bt_h, st_h, cj_h, ir_h, gd_h)

        def b_start_in(b, kk):
            off = (wid + NW * kk) * B_CHUNK
            for j, h in enumerate(b_idx_h):
                pltpu.async_copy(h.at[pl.ds(off, B_CHUNK)], bi_v[b][j],
                                 s_bi[b])

        def b_wait_in(b):
            for j, h in enumerate(b_idx_h):
                pltpu.make_async_copy(h.at[pl.ds(0, B_CHUNK)], bi_v[b][j],
                                      s_bi[b]).wait()

        def b_compute(b):
            @pl.loop(0, B_CHUNK // L)
            def _(g):
                s = pl.ds(g * L, L)
                v_bt = bi_v[b][0][s]
                v_st = bi_v[b][1][s]
                v_cj = bi_v[b][2][s]
                v_ir = bi_v[b][3][s]
                v_gd = bi_v[b][4][s]
                rb = (((v_bt * 8 + v_st) * 2 + v_cj) * 2 + v_ir) * 32 + v_gd
                rb_v[b][s] = rb

        def b_start_gather(b):
            pltpu.async_copy(FB_h.at[rb_v[b]], bo_v.at[b], s_bg[b])

        def b_wait_gather(b):
            pltpu.make_async_copy(FB_h.at[rb_v[b]], bo_v.at[b],
                                  s_bg[b]).wait()

        def b_post(b):
            @pl.loop(0, B_CHUNK)
            def _(r):
                for c in range(D_BOND // L):
                    cs = pl.ds(c * L, L)
                    bc_v[b, r, cs] = bo_v[b, r, cs]

        def b_start_out(b, kk):
            off = (wid + NW * kk) * B_CHUNK
            pltpu.async_copy(bc_v.at[b], bond_out.at[pl.ds(off, B_CHUNK)],
                             s_bo[b])

        def b_wait_out(b):
            pltpu.make_async_copy(bc_v.at[b], bond_out.at[pl.ds(0, B_CHUNK)],
                                  s_bo[b]).wait()

        n_b = (B_NCHUNK - wid + NW - 1) // NW
        run_pipeline(n_b, b_start_in, b_wait_in, b_compute, b_start_gather,
                     b_wait_gather, b_post, b_start_out, b_wait_out)

    return k(at, fc, nh, ar, hy, ch, bt, st, cj, ir, gd, FB, FA5, Wat)


def kernel(atom_type, formal_charge, num_H, aromaticity, hybridization,
           chiral, bond_type, stereo, conjugated, in_ring, graph_distance,
           W_atom_type, W_formal_charge, W_num_H, W_aromaticity,
           W_hybridization, W_chiral, W_bond_type, W_stereo, W_conjugated,
           W_in_ring, W_graph_distance):
    FB, FA5 = _build_fused_tables(
        W_atom_type, W_formal_charge, W_num_H, W_aromaticity,
        W_hybridization, W_chiral, W_bond_type, W_stereo, W_conjugated,
        W_in_ring, W_graph_distance)
    atom_emb, bond_emb = _sc_lookup(
        atom_type, formal_charge, num_H, aromaticity, hybridization, chiral,
        bond_type, stereo, conjugated, in_ring, graph_distance,
        FB, FA5, W_atom_type)
    return (atom_emb, bond_emb)


# speedup vs baseline: 10.9187x; 1.0221x over previous
import functools

import jax
import jax.numpy as jnp
from jax import lax
from jax.experimental import pallas as pl
from jax.experimental.pallas import tpu as pltpu
from jax.experimental.pallas import tpu_sc as plsc

N_ATOMS = 50000
N_BONDS = 800000
D_ATOM = 128
D_BOND = 64

NC = 2
NS = 16
NW = NC * NS
L = 16

A_CHUNK = 80
B_CHUNK = 128
A_NCHUNK = N_ATOMS // A_CHUNK
B_NCHUNK = N_BONDS // B_CHUNK

_f32 = jnp.float32
_i32 = jnp.int32


def _expand(w, inner, outer):
    return jnp.tile(jnp.repeat(w, inner, axis=0), (outer, 1))


def _build_fused_tables(Wat, Wfc, WnH, War, Why, Wch, Wbt, Wst, Wcj, Wir,
                        Wgd):
    def body(b, s, c, i, g, fc, nh, ar, hy, ch, fb, fa5):
        left = b[...] + s[...] + c[...] + i[...] + g[...]
        fb[...] = jnp.concatenate(
            [left, jnp.zeros((8192, D_BOND), _f32)], axis=1)
        fa5[...] = fc[...] + nh[...] + ar[...] + hy[...] + ch[...]

    b_e = _expand(Wbt, 1024, 1)
    s_e = _expand(Wst, 128, 8)
    c_e = _expand(Wcj, 64, 64)
    i_e = _expand(Wir, 32, 128)
    g_e = _expand(Wgd, 1, 256)
    fc_e = _expand(Wfc, 576, 1)
    nh_e = _expand(WnH, 64, 8)
    ar_e = _expand(War, 32, 72)
    hy_e = _expand(Why, 4, 144)
    ch_e = _expand(Wch, 1, 1152)

    return pl.pallas_call(
        body,
        out_shape=[
            jax.ShapeDtypeStruct((8192, 2 * D_BOND), _f32),
            jax.ShapeDtypeStruct((4608, D_ATOM), _f32),
        ],
    )(b_e, s_e, c_e, i_e, g_e, fc_e, nh_e, ar_e, hy_e, ch_e)


def _sc_lookup(at, fc, nh, ar, hy, ch, bt, st, cj, ir, gd, FB, FA5, Wat):
    mesh = plsc.VectorSubcoreMesh(core_axis_name="c", subcore_axis_name="s")

    @functools.partial(
        pl.kernel,
        out_type=[
            jax.ShapeDtypeStruct((N_ATOMS, D_ATOM), _f32),
            jax.ShapeDtypeStruct((N_BONDS, D_BOND), _f32),
        ],
        mesh=mesh,
        compiler_params=pltpu.CompilerParams(needs_layout_passes=False),
        scratch_types=[
            [[pltpu.VMEM((A_CHUNK,), _i32) for _ in range(6)]
             for _ in range(2)],
            [pltpu.VMEM((A_CHUNK,), _i32) for _ in range(2)],
            pltpu.VMEM((2, A_CHUNK, D_ATOM), _f32),
            pltpu.VMEM((2, A_CHUNK, D_ATOM), _f32),
            [[pltpu.VMEM((B_CHUNK,), _i32) for _ in range(5)]
             for _ in range(2)],
            [pltpu.VMEM((B_CHUNK,), _i32) for _ in range(2)],
            pltpu.VMEM((2, B_CHUNK, 2 * D_BOND), _f32),
            pltpu.VMEM((2, B_CHUNK, D_BOND), _f32),
            [pltpu.SemaphoreType.DMA, pltpu.SemaphoreType.DMA],
            [pltpu.SemaphoreType.DMA, pltpu.SemaphoreType.DMA],
            [pltpu.SemaphoreType.DMA, pltpu.SemaphoreType.DMA],
            [pltpu.SemaphoreType.DMA, pltpu.SemaphoreType.DMA],
            [pltpu.SemaphoreType.DMA, pltpu.SemaphoreType.DMA],
            [pltpu.SemaphoreType.DMA, pltpu.SemaphoreType.DMA],
        ],
    )
    def k(at_h, fc_h, nh_h, ar_h, hy_h, ch_h,
          bt_h, st_h, cj_h, ir_h, gd_h,
          FB_h, FA5_h, Wat_h,
          atom_out, bond_out,
          ai_v, ra_v, ao_v, ab_v, bi_v, rb_v, bo_v, bc_v,
          s_ai, s_ag, s_ao, s_bi, s_bg, s_bo):
        wid = lax.axis_index("s") * NC + lax.axis_index("c")

        def run_pipeline(n, start_in, wait_in, compute, start_gather,
                         wait_gather, post, start_out, wait_out):
            @pl.when(n >= 1)
            def _():
                start_in(0, 0)

            @pl.loop(0, n, step=2)
            def _(k0):
                for b in range(2):
                    kk = k0 + b
                    o = 1 - b

                    @pl.when(kk < n)
                    def _():
                        wait_in(b)
                        compute(b)

                        @pl.when(kk >= 1)
                        def _():
                            wait_gather(o)
                            post(o)
                            start_out(o, kk - 1)

                        @pl.when(kk + 1 < n)
                        def _():
                            start_in(o, kk + 1)

                        @pl.when(kk >= 2)
                        def _():
                            wait_out(b)

                        start_gather(b)

            last = (n - 1) % 2

            @pl.when(n >= 1)
            def _():
                for b in range(2):
                    @pl.when(last == b)
                    def _():
                        wait_gather(b)
                        post(b)
                        start_out(b, n - 1)

                wait_out(0)

            @pl.when(n >= 2)
            def _():
                wait_out(1)

        a_idx_h = (at_h, fc_h, nh_h, ar_h, hy_h, ch_h)

        def a_start_in(b, kk):
            off = (wid + NW * kk) * A_CHUNK
            for j, h in enumerate(a_idx_h):
                pltpu.async_copy(h.at[pl.ds(off, A_CHUNK)], ai_v[b][j],
                                 s_ai[b])

        def a_wait_in(b):
            for j, h in enumerate(a_idx_h):
                pltpu.make_async_copy(h.at[pl.ds(0, A_CHUNK)], ai_v[b][j],
                                      s_ai[b]).wait()

        def a_compute(b):
            @pl.loop(0, A_CHUNK // L)
            def _(g):
                s = pl.ds(g * L, L)
                v_fc = ai_v[b][1][s]
                v_nh = ai_v[b][2][s]
                v_ar = ai_v[b][3][s]
                v_hy = ai_v[b][4][s]
                v_ch = ai_v[b][5][s]
                ra = ((((v_fc + 1) * 9 + v_nh) * 2 + v_ar) * 8
                      + v_hy) * 4 + v_ch
                ra_v[b][s] = ra

        def a_start_gather(b):
            pltpu.async_copy(FA5_h.at[ra_v[b]], ao_v.at[b], s_ag[b])
            pltpu.async_copy(Wat_h.at[ai_v[b][0]], ab_v.at[b], s_ag[b])

        def a_wait_gather(b):
            pltpu.make_async_copy(FA5_h.at[ra_v[b]], ao_v.at[b],
                                  s_ag[b]).wait()
            pltpu.make_async_copy(Wat_h.at[ai_v[b][0]], ab_v.at[b],
                                  s_ag[b]).wait()

        def a_post(b):
            @pl.loop(0, A_CHUNK)
            def _(r):
                for c in range(D_ATOM // L):
                    cs = pl.ds(c * L, L)
                    plsc.addupdate(ao_v.at[b, r, cs], ab_v[b, r, cs])

        def a_start_out(b, kk):
            off = (wid + NW * kk) * A_CHUNK
            pltpu.async_copy(ao_v.at[b], atom_out.at[pl.ds(off, A_CHUNK)],
                             s_ao[b])

        def a_wait_out(b):
            pltpu.make_async_copy(ao_v.at[b], atom_out.at[pl.ds(0, A_CHUNK)],
                                  s_ao[b]).wait()

        n_a = (A_NCHUNK - wid + NW - 1) // NW
        run_pipeline(n_a, a_start_in, a_wait_in, a_compute, a_start_gather,
                     a_wait_gather, a_post, a_start_out, a_wait_out)

        b_idx_h = (bt_h, st_h, cj_h, ir_h, gd_h)

        def b_start_in(b, kk):
            off = (wid + NW * kk) * B_CHUNK
            for j, h in enumerate(b_idx_h):
                pltpu.async_copy(h.at[pl.ds(off, B_CHUNK)], bi_v[b][j],
                                 s_bi[b])

        def b_wait_in(b):
            for j, h in enumerate(b_idx_h):
                pltpu.make_async_copy(h.at[pl.ds(0, B_CHUNK)], bi_v[b][j],
                                      s_bi[b]).wait()

        def b_compute(b):
            @pl.loop(0, B_CHUNK // L)
            def _(g):
                s = pl.ds(g * L, L)
                v_bt = bi_v[b][0][s]
                v_st = bi_v[b][1][s]
                v_cj = bi_v[b][2][s]
                v_ir = bi_v[b][3][s]
                v_gd = bi_v[b][4][s]
                rb = (((v_bt * 8 + v_st) * 2 + v_cj) * 2 + v_ir) * 32 + v_gd
                rb_v[b][s] = rb

        def b_start_gather(b):
            pltpu.async_copy(FB_h.at[rb_v[b]], bo_v.at[b], s_bg[b])

        def b_wait_gather(b):
            pltpu.make_async_copy(FB_h.at[rb_v[b]], bo_v.at[b],
                                  s_bg[b]).wait()

        def b_post(b):
            @pl.loop(0, B_CHUNK)
            def _(r):
                for c in range(D_BOND // L):
                    cs = pl.ds(c * L, L)
                    bc_v[b, r, cs] = bo_v[b, r, cs]

        def b_start_out(b, kk):
            off = (wid + NW * kk) * B_CHUNK
            pltpu.async_copy(bc_v.at[b], bond_out.at[pl.ds(off, B_CHUNK)],
                             s_bo[b])

        def b_wait_out(b):
            pltpu.make_async_copy(bc_v.at[b], bond_out.at[pl.ds(0, B_CHUNK)],
                                  s_bo[b]).wait()

        n_b = (B_NCHUNK - wid + NW - 1) // NW
        run_pipeline(n_b, b_start_in, b_wait_in, b_compute, b_start_gather,
                     b_wait_gather, b_post, b_start_out, b_wait_out)

    return k(at, fc, nh, ar, hy, ch, bt, st, cj, ir, gd, FB, FA5, Wat)


def kernel(atom_type, formal_charge, num_H, aromaticity, hybridization,
           chiral, bond_type, stereo, conjugated, in_ring, graph_distance,
           W_atom_type, W_formal_charge, W_num_H, W_aromaticity,
           W_hybridization, W_chiral, W_bond_type, W_stereo, W_conjugated,
           W_in_ring, W_graph_distance):
    FB, FA5 = _build_fused_tables(
        W_atom_type, W_formal_charge, W_num_H, W_aromaticity,
        W_hybridization, W_chiral, W_bond_type, W_stereo, W_conjugated,
        W_in_ring, W_graph_distance)
    atom_emb, bond_emb = _sc_lookup(
        atom_type, formal_charge, num_H, aromaticity, hybridization, chiral,
        bond_type, stereo, conjugated, in_ring, graph_distance,
        FB, FA5, W_atom_type)
    return (atom_emb, bond_emb)


# speedup vs baseline: 11.6907x; 1.0707x over previous
import functools

import jax
import jax.numpy as jnp
from jax import lax
from jax.experimental import pallas as pl
from jax.experimental.pallas import tpu as pltpu
from jax.experimental.pallas import tpu_sc as plsc

N_ATOMS = 50000
N_BONDS = 800000
D_ATOM = 128
D_BOND = 64

NC = 2
NS = 16
NW = NC * NS
L = 16

A_CHUNK = 80
B_CHUNK = 64
B_SLOTS = 4
A_NCHUNK = N_ATOMS // A_CHUNK
B_NCHUNK = N_BONDS // B_CHUNK

_f32 = jnp.float32
_i32 = jnp.int32


def _expand(w, inner, outer):
    return jnp.tile(jnp.repeat(w, inner, axis=0), (outer, 1))


def _build_fused_tables(Wat, Wfc, WnH, War, Why, Wch, Wbt, Wst, Wcj, Wir,
                        Wgd):
    def body(b, s, c, i, g, fc, nh, ar, hy, ch, fb, fa5):
        left = b[...] + s[...] + c[...] + i[...] + g[...]
        fb[...] = jnp.concatenate(
            [left, jnp.zeros((8192, D_BOND), _f32)], axis=1)
        fa5[...] = fc[...] + nh[...] + ar[...] + hy[...] + ch[...]

    b_e = _expand(Wbt, 1024, 1)
    s_e = _expand(Wst, 128, 8)
    c_e = _expand(Wcj, 64, 64)
    i_e = _expand(Wir, 32, 128)
    g_e = _expand(Wgd, 1, 256)
    fc_e = _expand(Wfc, 576, 1)
    nh_e = _expand(WnH, 64, 8)
    ar_e = _expand(War, 32, 72)
    hy_e = _expand(Why, 4, 144)
    ch_e = _expand(Wch, 1, 1152)

    return pl.pallas_call(
        body,
        out_shape=[
            jax.ShapeDtypeStruct((8192, 2 * D_BOND), _f32),
            jax.ShapeDtypeStruct((4608, D_ATOM), _f32),
        ],
    )(b_e, s_e, c_e, i_e, g_e, fc_e, nh_e, ar_e, hy_e, ch_e)


def _sc_lookup(at, fc, nh, ar, hy, ch, bt, st, cj, ir, gd, FB, FA5, Wat):
    mesh = plsc.VectorSubcoreMesh(core_axis_name="c", subcore_axis_name="s")

    @functools.partial(
        pl.kernel,
        out_type=[
            jax.ShapeDtypeStruct((N_ATOMS, D_ATOM), _f32),
            jax.ShapeDtypeStruct((N_BONDS, D_BOND), _f32),
        ],
        mesh=mesh,
        compiler_params=pltpu.CompilerParams(needs_layout_passes=False),
        scratch_types=[
            [[pltpu.VMEM((A_CHUNK,), _i32) for _ in range(6)]
             for _ in range(2)],
            [pltpu.VMEM((A_CHUNK,), _i32) for _ in range(2)],
            pltpu.VMEM((2, A_CHUNK, D_ATOM), _f32),
            pltpu.VMEM((2, A_CHUNK, D_ATOM), _f32),
            [[pltpu.VMEM((B_CHUNK,), _i32) for _ in range(5)]
             for _ in range(B_SLOTS)],
            [pltpu.VMEM((B_CHUNK,), _i32) for _ in range(B_SLOTS)],
            pltpu.VMEM((B_SLOTS, B_CHUNK, 2 * D_BOND), _f32),
            pltpu.VMEM((B_SLOTS, B_CHUNK, D_BOND), _f32),
            [pltpu.SemaphoreType.DMA, pltpu.SemaphoreType.DMA],
            [pltpu.SemaphoreType.DMA, pltpu.SemaphoreType.DMA],
            [pltpu.SemaphoreType.DMA, pltpu.SemaphoreType.DMA],
            [pltpu.SemaphoreType.DMA for _ in range(B_SLOTS)],
            [pltpu.SemaphoreType.DMA for _ in range(B_SLOTS)],
            [pltpu.SemaphoreType.DMA for _ in range(B_SLOTS)],
        ],
    )
    def k(at_h, fc_h, nh_h, ar_h, hy_h, ch_h,
          bt_h, st_h, cj_h, ir_h, gd_h,
          FB_h, FA5_h, Wat_h,
          atom_out, bond_out,
          ai_v, ra_v, ao_v, ab_v, bi_v, rb_v, bo_v, bc_v,
          s_ai, s_ag, s_ao, s_bi, s_bg, s_bo):
        wid = lax.axis_index("s") * NC + lax.axis_index("c")

        def run_pipeline(ns, n, start_in, wait_in, compute, start_gather,
                         wait_gather, post, start_out, wait_out):
            @pl.when(n >= 1)
            def _():
                start_in(0, 0)

            @pl.loop(0, n + ns - 1, step=ns)
            def _(k0):
                for b in range(ns):
                    kk = k0 + b
                    q = (b + 1) % ns
                    m = kk - (ns - 1)

                    @pl.when(kk < n)
                    def _():
                        wait_in(b)
                        compute(b)

                    @pl.when((m >= 0) & (m < n))
                    def _():
                        wait_gather(q)
                        post(q)
                        start_out(q, m)

                    @pl.when(kk + 1 < n)
                    def _():
                        start_in(q, kk + 1)

                    @pl.when((kk >= ns) & (kk < n))
                    def _():
                        wait_out(b)

                    @pl.when(kk < n)
                    def _():
                        start_gather(b)

            for s2 in range(ns):
                @pl.when(n > s2)
                def _():
                    wait_out(s2)

        a_idx_h = (at_h, fc_h, nh_h, ar_h, hy_h, ch_h)

        def a_start_in(b, kk):
            off = (wid + NW * kk) * A_CHUNK
            for j, h in enumerate(a_idx_h):
                pltpu.async_copy(h.at[pl.ds(off, A_CHUNK)], ai_v[b][j],
                                 s_ai[b])

        def a_wait_in(b):
            for j, h in enumerate(a_idx_h):
                pltpu.make_async_copy(h.at[pl.ds(0, A_CHUNK)], ai_v[b][j],
                                      s_ai[b]).wait()

        def a_compute(b):
            @pl.loop(0, A_CHUNK // L)
            def _(g):
                s = pl.ds(g * L, L)
                v_fc = ai_v[b][1][s]
                v_nh = ai_v[b][2][s]
                v_ar = ai_v[b][3][s]
                v_hy = ai_v[b][4][s]
                v_ch = ai_v[b][5][s]
                ra = ((((v_fc + 1) * 9 + v_nh) * 2 + v_ar) * 8
                      + v_hy) * 4 + v_ch
                ra_v[b][s] = ra

        def a_start_gather(b):
            pltpu.async_copy(FA5_h.at[ra_v[b]], ao_v.at[b], s_ag[b])
            pltpu.async_copy(Wat_h.at[ai_v[b][0]], ab_v.at[b], s_ag[b])

        def a_wait_gather(b):
            pltpu.make_async_copy(FA5_h.at[ra_v[b]], ao_v.at[b],
                                  s_ag[b]).wait()
            pltpu.make_async_copy(Wat_h.at[ai_v[b][0]], ab_v.at[b],
                                  s_ag[b]).wait()

        def a_post(b):
            @pl.loop(0, A_CHUNK)
            def _(r):
                for c in range(D_ATOM // L):
                    cs = pl.ds(c * L, L)
                    plsc.addupdate(ao_v.at[b, r, cs], ab_v[b, r, cs])

        def a_start_out(b, kk):
            off = (wid + NW * kk) * A_CHUNK
            pltpu.async_copy(ao_v.at[b], atom_out.at[pl.ds(off, A_CHUNK)],
                             s_ao[b])

        def a_wait_out(b):
            pltpu.make_async_copy(ao_v.at[b], atom_out.at[pl.ds(0, A_CHUNK)],
                                  s_ao[b]).wait()

        n_a = (A_NCHUNK - wid + NW - 1) // NW
        run_pipeline(2, n_a, a_start_in, a_wait_in, a_compute,
                     a_start_gather, a_wait_gather, a_post, a_start_out,
                     a_wait_out)

        b_idx_h = (bt_h, st_h, cj_h, ir_h, gd_h)

        def b_start_in(b, kk):
            off = (wid + NW * kk) * B_CHUNK
            for j, h in enumerate(b_idx_h):
                pltpu.async_copy(h.at[pl.ds(off, B_CHUNK)], bi_v[b][j],
                                 s_bi[b])

        def b_wait_in(b):
            for j, h in enumerate(b_idx_h):
                pltpu.make_async_copy(h.at[pl.ds(0, B_CHUNK)], bi_v[b][j],
                                      s_bi[b]).wait()

        def b_compute(b):
            @pl.loop(0, B_CHUNK // L)
            def _(g):
                s = pl.ds(g * L, L)
                v_bt = bi_v[b][0][s]
                v_st = bi_v[b][1][s]
                v_cj = bi_v[b][2][s]
                v_ir = bi_v[b][3][s]
                v_gd = bi_v[b][4][s]
                rb = (((v_bt * 8 + v_st) * 2 + v_cj) * 2 + v_ir) * 32 + v_gd
                rb_v[b][s] = rb

        def b_start_gather(b):
            pltpu.async_copy(FB_h.at[rb_v[b]], bo_v.at[b], s_bg[b])

        def b_wait_gather(b):
            pltpu.make_async_copy(FB_h.at[rb_v[b]], bo_v.at[b],
                                  s_bg[b]).wait()

        def b_post(b):
            @pl.loop(0, B_CHUNK)
            def _(r):
                for c in range(D_BOND // L):
                    cs = pl.ds(c * L, L)
                    bc_v[b, r, cs] = bo_v[b, r, cs]

        def b_start_out(b, kk):
            off = (wid + NW * kk) * B_CHUNK
            pltpu.async_copy(bc_v.at[b], bond_out.at[pl.ds(off, B_CHUNK)],
                             s_bo[b])

        def b_wait_out(b):
            pltpu.make_async_copy(bc_v.at[b], bond_out.at[pl.ds(0, B_CHUNK)],
                                  s_bo[b]).wait()

        n_b = (B_NCHUNK - wid + NW - 1) // NW
        run_pipeline(B_SLOTS, n_b, b_start_in, b_wait_in, b_compute,
                     b_start_gather, b_wait_gather, b_post, b_start_out,
                     b_wait_out)

    return k(at, fc, nh, ar, hy, ch, bt, st, cj, ir, gd, FB, FA5, Wat)


def kernel(atom_type, formal_charge, num_H, aromaticity, hybridization,
           chiral, bond_type, stereo, conjugated, in_ring, graph_distance,
           W_atom_type, W_formal_charge, W_num_H, W_aromaticity,
           W_hybridization, W_chiral, W_bond_type, W_stereo, W_conjugated,
           W_in_ring, W_graph_distance):
    FB, FA5 = _build_fused_tables(
        W_atom_type, W_formal_charge, W_num_H, W_aromaticity,
        W_hybridization, W_chiral, W_bond_type, W_stereo, W_conjugated,
        W_in_ring, W_graph_distance)
    atom_emb, bond_emb = _sc_lookup(
        atom_type, formal_charge, num_H, aromaticity, hybridization, chiral,
        bond_type, stereo, conjugated, in_ring, graph_distance,
        FB, FA5, W_atom_type)
    return (atom_emb, bond_emb)


# speedup vs baseline: 12.7833x; 1.0935x over previous
import functools

import jax
import jax.numpy as jnp
from jax import lax
from jax.experimental import pallas as pl
from jax.experimental.pallas import tpu as pltpu
from jax.experimental.pallas import tpu_sc as plsc

N_ATOMS = 50000
N_BONDS = 800000
D_ATOM = 128
D_BOND = 64

NC = 2
NS = 16
NW = NC * NS
L = 16

A_CHUNK = 80
B_CHUNK = 64
A_NCHUNK = N_ATOMS // A_CHUNK
B_NCHUNK = N_BONDS // B_CHUNK

_f32 = jnp.float32
_i32 = jnp.int32


def _expand(w, inner, outer):
    return jnp.tile(jnp.repeat(w, inner, axis=0), (outer, 1))


def _build_fused_tables(Wat, Wfc, WnH, War, Why, Wch, Wbt, Wst, Wcj, Wir,
                        Wgd):
    def body(b, s, c, i, g, b2, s2, c2, i2, fc, nh, ar, hy, ch, fb, fbc,
             fa5):
        zb = jnp.zeros((8192, D_BOND), _f32)
        fb[...] = jnp.concatenate(
            [b[...] + s[...] + c[...] + i[...] + g[...], zb], axis=1)
        fbc[...] = b2[...] + s2[...] + c2[...] + i2[...]
        fa5[...] = fc[...] + nh[...] + ar[...] + hy[...] + ch[...]

    b_e = _expand(Wbt, 1024, 1)
    s_e = _expand(Wst, 128, 8)
    c_e = _expand(Wcj, 64, 64)
    i_e = _expand(Wir, 32, 128)
    g_e = _expand(Wgd, 1, 256)
    gd_pad = jnp.pad(Wgd, ((0, 256 - 32), (0, 0)))
    b2_e = jnp.concatenate([_expand(Wbt, 32, 1), gd_pad], axis=1)
    zp = jnp.zeros((256, D_BOND), _f32)
    s2_e = jnp.concatenate([_expand(Wst, 4, 8), zp], axis=1)
    c2_e = jnp.concatenate([_expand(Wcj, 2, 64), zp], axis=1)
    i2_e = jnp.concatenate([_expand(Wir, 1, 128), zp], axis=1)
    fc_e = _expand(Wfc, 576, 1)
    nh_e = _expand(WnH, 64, 8)
    ar_e = _expand(War, 32, 72)
    hy_e = _expand(Why, 4, 144)
    ch_e = _expand(Wch, 1, 1152)

    return pl.pallas_call(
        body,
        out_shape=[
            jax.ShapeDtypeStruct((8192, 2 * D_BOND), _f32),
            jax.ShapeDtypeStruct((256, 2 * D_BOND), _f32),
            jax.ShapeDtypeStruct((4608, D_ATOM), _f32),
        ],
    )(b_e, s_e, c_e, i_e, g_e, b2_e, s2_e, c2_e, i2_e, fc_e, nh_e, ar_e,
      hy_e, ch_e)


def _sc_lookup(at, fc, nh, ar, hy, ch, bt, st, cj, ir, gd, FB, FBC, FA5,
               Wat):
    mesh = plsc.VectorSubcoreMesh(core_axis_name="c", subcore_axis_name="s")

    @functools.partial(
        pl.kernel,
        out_type=[
            jax.ShapeDtypeStruct((N_ATOMS, D_ATOM), _f32),
            jax.ShapeDtypeStruct((N_BONDS, D_BOND), _f32),
        ],
        mesh=mesh,
        compiler_params=pltpu.CompilerParams(needs_layout_passes=False),
        scratch_types=[
            pltpu.VMEM((256, 2 * D_BOND), _f32),
            pltpu.VMEM((100, D_ATOM), _f32),
            [[pltpu.VMEM((A_CHUNK,), _i32) for _ in range(6)]
             for _ in range(2)],
            [pltpu.VMEM((A_CHUNK,), _i32) for _ in range(2)],
            pltpu.VMEM((2, A_CHUNK, D_ATOM), _f32),
            [pltpu.VMEM((5 * B_CHUNK,), _i32) for _ in range(2)],
            [pltpu.VMEM((B_CHUNK,), _i32) for _ in range(2)],
            pltpu.VMEM((2, B_CHUNK, 2 * D_BOND), _f32),
            pltpu.VMEM((2, B_CHUNK, D_BOND), _f32),
            [pltpu.VMEM((5 * B_CHUNK,), _i32) for _ in range(2)],
            pltpu.VMEM((2, B_CHUNK, D_BOND), _f32),
            [pltpu.SemaphoreType.DMA, pltpu.SemaphoreType.DMA],
            [pltpu.SemaphoreType.DMA, pltpu.SemaphoreType.DMA],
            [pltpu.SemaphoreType.DMA, pltpu.SemaphoreType.DMA],
            [pltpu.SemaphoreType.DMA, pltpu.SemaphoreType.DMA],
            [pltpu.SemaphoreType.DMA, pltpu.SemaphoreType.DMA],
            [pltpu.SemaphoreType.DMA, pltpu.SemaphoreType.DMA],
            [pltpu.SemaphoreType.DMA, pltpu.SemaphoreType.DMA],
            [pltpu.SemaphoreType.DMA, pltpu.SemaphoreType.DMA],
        ],
    )
    def k(at_h, fc_h, nh_h, ar_h, hy_h, ch_h,
          bt_h, st_h, cj_h, ir_h, gd_h,
          FB_h, FBC_h, FA5_h, Wat_h,
          atom_out, bond_out,
          fbc_v, wat_v, ai_v, ra_v, ao_v, bsi_v, rb_v, bo_v, bc_v,
          bti_v, tbo_v,
          s_ai, s_ag, s_ao, s_bi, s_bg, s_bo, s_ti, s_to):
        wid = lax.axis_index("s") * NC + lax.axis_index("c")

        pltpu.sync_copy(FBC_h, fbc_v)
        pltpu.sync_copy(Wat_h, wat_v)

        b_idx_h = (bt_h, st_h, cj_h, ir_h, gd_h)
        a_idx_h = (at_h, fc_h, nh_h, ar_h, hy_h, ch_h)

        def run_pipeline(ns, n, start_in, wait_in, compute, start_gather,
                         wait_gather, post, start_out, wait_out, extra):
            @pl.when(n >= 1)
            def _():
                start_in(0, 0)

            @pl.loop(0, n + ns - 1, step=ns)
            def _(k0):
                for b in range(ns):
                    kk = k0 + b
                    q = (b + 1) % ns
                    m = kk - (ns - 1)

                    @pl.when(kk < n)
                    def _():
                        wait_in(b)
                        compute(b)
                        start_gather(b)

                    extra(kk, b)

                    @pl.when((m >= 0) & (m < n))
                    def _():
                        wait_gather(q)
                        post(q)
                        start_out(q, m)

                    @pl.when(kk + 1 < n)
                    def _():
                        start_in(q, kk + 1)

                    @pl.when((kk >= ns) & (kk < n))
                    def _():
                        wait_out(b)

            for s2 in range(ns):
                @pl.when(n > s2)
                def _():
                    wait_out(s2)

        def a_start_in(b, kk):
            off = (wid + NW * kk) * A_CHUNK
            for j, h in enumerate(a_idx_h):
                pltpu.async_copy(h.at[pl.ds(off, A_CHUNK)], ai_v[b][j],
                                 s_ai[b])

        def a_wait_in(b):
            for j, h in enumerate(a_idx_h):
                pltpu.make_async_copy(h.at[pl.ds(0, A_CHUNK)], ai_v[b][j],
                                      s_ai[b]).wait()

        def a_compute(b):
            @pl.loop(0, A_CHUNK // L)
            def _(g):
                s = pl.ds(g * L, L)
                v_fc = ai_v[b][1][s]
                v_nh = ai_v[b][2][s]
                v_ar = ai_v[b][3][s]
                v_hy = ai_v[b][4][s]
                v_ch = ai_v[b][5][s]
                ra = ((((v_fc + 1) * 9 + v_nh) * 2 + v_ar) * 8
                      + v_hy) * 4 + v_ch
                ra_v[b][s] = ra

        def a_start_gather(b):
            pltpu.async_copy(FA5_h.at[ra_v[b]], ao_v.at[b], s_ag[b])

        def a_wait_gather(b):
            pltpu.make_async_copy(FA5_h.at[ra_v[b]], ao_v.at[b],
                                  s_ag[b]).wait()

        def a_post(b):
            @plsc.parallel_loop(0, A_CHUNK // L, unroll=2)
            def _(g):
                s = g * L
                v_at = ai_v[b][0][pl.ds(s, L)]
                for j in range(L):
                    aj = v_at[j]
                    for c in range(D_ATOM // L):
                        cs = pl.ds(c * L, L)
                        plsc.addupdate(ao_v.at[b, s + j, cs], wat_v[aj, cs])

        def a_start_out(b, kk):
            off = (wid + NW * kk) * A_CHUNK
            pltpu.async_copy(ao_v.at[b], atom_out.at[pl.ds(off, A_CHUNK)],
                             s_ao[b])

        def a_wait_out(b):
            pltpu.make_async_copy(ao_v.at[b], atom_out.at[pl.ds(0, A_CHUNK)],
                                  s_ao[b]).wait()

        n_a = (A_NCHUNK - wid + NW - 1) // NW
        run_pipeline(2, n_a, a_start_in, a_wait_in, a_compute,
                     a_start_gather, a_wait_gather, a_post, a_start_out,
                     a_wait_out, lambda kk, b: None)

        n = (B_NCHUNK - wid + NW - 1) // NW
        n_s = (n + 1) // 2
        n_t = n - n_s

        def bs_start_in(b, kk):
            off = (wid + NW * kk) * B_CHUNK
            for j, h in enumerate(b_idx_h):
                pltpu.async_copy(h.at[pl.ds(off, B_CHUNK)],
                                 bsi_v[b].at[pl.ds(j * B_CHUNK, B_CHUNK)],
                                 s_bi[b])

        def bs_wait_in(b):
            for j, h in enumerate(b_idx_h):
                pltpu.make_async_copy(
                    h.at[pl.ds(0, B_CHUNK)],
                    bsi_v[b].at[pl.ds(j * B_CHUNK, B_CHUNK)],
                    s_bi[b]).wait()

        def bs_compute(b):
            @pl.loop(0, B_CHUNK // L)
            def _(g):
                s = g * L
                v_bt = bsi_v[b][pl.ds(s, L)]
                v_st = bsi_v[b][pl.ds(B_CHUNK + s, L)]
                v_cj = bsi_v[b][pl.ds(2 * B_CHUNK + s, L)]
                v_ir = bsi_v[b][pl.ds(3 * B_CHUNK + s, L)]
                v_gd = bsi_v[b][pl.ds(4 * B_CHUNK + s, L)]
                rb = (((v_bt * 8 + v_st) * 2 + v_cj) * 2 + v_ir) * 32 + v_gd
                rb_v[b][pl.ds(s, L)] = rb

        def bs_start_gather(b):
            pltpu.async_copy(FB_h.at[rb_v[b]], bo_v.at[b], s_bg[b])

        def bs_wait_gather(b):
            pltpu.make_async_copy(FB_h.at[rb_v[b]], bo_v.at[b],
                                  s_bg[b]).wait()

        def bs_post(b):
            @plsc.parallel_loop(0, B_CHUNK, unroll=2)
            def _(r):
                for c in range(D_BOND // L):
                    cs = pl.ds(c * L, L)
                    bc_v[b, r, cs] = bo_v[b, r, cs]

        def bs_start_out(b, kk):
            off = (wid + NW * kk) * B_CHUNK
            pltpu.async_copy(bc_v.at[b], bond_out.at[pl.ds(off, B_CHUNK)],
                             s_bo[b])

        def bs_wait_out(b):
            pltpu.make_async_copy(bc_v.at[b], bond_out.at[pl.ds(0, B_CHUNK)],
                                  s_bo[b]).wait()

        def bt_start_in(b, kk):
            off = (wid + NW * (n_s + kk)) * B_CHUNK
            for j, h in enumerate(b_idx_h):
                pltpu.async_copy(h.at[pl.ds(off, B_CHUNK)],
                                 bti_v[b].at[pl.ds(j * B_CHUNK, B_CHUNK)],
                                 s_ti[b])

        def bt_wait_in(b):
            for j, h in enumerate(b_idx_h):
                pltpu.make_async_copy(
                    h.at[pl.ds(0, B_CHUNK)],
                    bti_v[b].at[pl.ds(j * B_CHUNK, B_CHUNK)],
                    s_ti[b]).wait()

        def bt_compute(b):
            @plsc.parallel_loop(0, B_CHUNK // L, unroll=2)
            def _(g):
                s = g * L
                v_bt = bti_v[b][pl.ds(s, L)]
                v_st = bti_v[b][pl.ds(B_CHUNK + s, L)]
                v_cj = bti_v[b][pl.ds(2 * B_CHUNK + s, L)]
                v_ir = bti_v[b][pl.ds(3 * B_CHUNK + s, L)]
                v_gd = bti_v[b][pl.ds(4 * B_CHUNK + s, L)]
                r1 = ((v_bt * 8 + v_st) * 2 + v_cj) * 2 + v_ir
                for j in range(L):
                    r1j, gdj = r1[j], v_gd[j]
                    for c in range(D_BOND // L):
                        cs = pl.ds(c * L, L)
                        tbo_v[b, s + j, cs] = (
                            fbc_v[r1j, cs]
                            + fbc_v[gdj, pl.ds(D_BOND + c * L, L)])

        def bt_start_out(b, kk):
            off = (wid + NW * (n_s + kk)) * B_CHUNK
            pltpu.async_copy(tbo_v.at[b], bond_out.at[pl.ds(off, B_CHUNK)],
                             s_to[b])

        def bt_wait_out(b):
            pltpu.make_async_copy(tbo_v.at[b],
                                  bond_out.at[pl.ds(0, B_CHUNK)],
                                  s_to[b]).wait()

        def bt_extra(kk, b):
            @pl.when(kk < n_t)
            def _():
                bt_wait_in(b)

                @pl.when(kk >= 2)
                def _():
                    bt_wait_out(b)

                bt_compute(b)
                bt_start_out(b, kk)

                @pl.when(kk + 1 < n_t)
                def _():
                    bt_start_in(1 - b, kk + 1)

        @pl.when(n_t >= 1)
        def _():
            bt_start_in(0, 0)

        run_pipeline(2, n_s, bs_start_in, bs_wait_in, bs_compute,
                     bs_start_gather, bs_wait_gather, bs_post, bs_start_out,
                     bs_wait_out, bt_extra)

        @pl.when(n_t >= 1)
        def _():
            bt_wait_out(0)

        @pl.when(n_t >= 2)
        def _():
            bt_wait_out(1)

    return k(at, fc, nh, ar, hy, ch, bt, st, cj, ir, gd, FB, FBC, FA5, Wat)


def kernel(atom_type, formal_charge, num_H, aromaticity, hybridization,
           chiral, bond_type, stereo, conjugated, in_ring, graph_distance,
           W_atom_type, W_formal_charge, W_num_H, W_aromaticity,
           W_hybridization, W_chiral, W_bond_type, W_stereo, W_conjugated,
           W_in_ring, W_graph_distance):
    FB, FBC, FA5 = _build_fused_tables(
        W_atom_type, W_formal_charge, W_num_H, W_aromaticity,
        W_hybridization, W_chiral, W_bond_type, W_stereo, W_conjugated,
        W_in_ring, W_graph_distance)
    atom_emb, bond_emb = _sc_lookup(
        atom_type, formal_charge, num_H, aromaticity, hybridization, chiral,
        bond_type, stereo, conjugated, in_ring, graph_distance,
        FB, FBC, FA5, W_atom_type)
    return (atom_emb, bond_emb)
